# Initial kernel scaffold; baseline (speedup 1.0000x reference)
import jax
import jax.numpy as jnp
from jax.experimental import pallas as pl


def kernel(x, edge_index, W_rel1, b_rel1, W_root1, W_rel2, b_rel2, W_root2, W_rel3, b_rel3, W_root3, W_lin, b_lin):
    raise NotImplementedError("write your pallas kernel here")


# speedup vs baseline: 4.9117x; 4.9117x over previous
import functools

import jax
import jax.numpy as jnp
from jax import lax
from jax.experimental import pallas as pl
from jax.experimental.pallas import tpu as pltpu
from jax.experimental.pallas import tpu_sc as plsc

N = 10000
E = 320000
D = 128
D_OUT = 64

NC = 2
NS = 16
NW = NC * NS
EPW = E // NW
CHUNK = 80
NCHUNK = EPW // CHUNK
ROWS_PER_TILE = 624
ZCOPIES = ROWS_PER_TILE // CHUNK
ZREM = ROWS_PER_TILE % CHUNK
TAIL0 = NS * ROWS_PER_TILE
TAIL = N - TAIL0


def _sc_agg_body(h_hbm, src_hbm, dst_hbm, out_hbm,
                 src_c, dst_c, rows, agg_sh, sem):
    c = lax.axis_index("c")
    s = lax.axis_index("s")
    wid = s * NC + c
    base = wid * EPW

    def zero_body(i, carry):
        r = i // (D // 16)
        col = (i % (D // 16)) * 16
        rows[r, pl.ds(col, 16)] = jnp.zeros((16,), jnp.float32)
        return carry

    lax.fori_loop(0, CHUNK * (D // 16), zero_body, 0)

    row0 = pl.multiple_of(s * ROWS_PER_TILE, 8)
    for k in range(ZCOPIES):
        pltpu.sync_copy(rows, agg_sh.at[pl.ds(row0 + k * CHUNK, CHUNK)])
    if ZREM:
        pltpu.sync_copy(rows.at[pl.ds(0, ZREM)],
                        agg_sh.at[pl.ds(row0 + ZCOPIES * CHUNK, ZREM)])

    @pl.when(s == NS - 1)
    def _zero_tail():
        pltpu.sync_copy(rows.at[pl.ds(0, TAIL)], agg_sh.at[pl.ds(TAIL0, TAIL)])

    plsc.subcore_barrier()

    def body(j, carry):
        off = pl.multiple_of(base + j * CHUNK, 8)
        pltpu.sync_copy(src_hbm.at[pl.ds(off, CHUNK)], src_c)
        pltpu.sync_copy(dst_hbm.at[pl.ds(off, CHUNK)], dst_c)
        pltpu.sync_copy(h_hbm.at[src_c], rows)
        pltpu.sync_copy(rows, agg_sh.at[dst_c], add=True)
        return carry

    lax.fori_loop(0, NCHUNK, body, 0)

    plsc.subcore_barrier()

    out_row = pl.multiple_of(c * N + row0, 8)
    pltpu.sync_copy(agg_sh.at[pl.ds(row0, ROWS_PER_TILE)],
                    out_hbm.at[pl.ds(out_row, ROWS_PER_TILE)])

    @pl.when(s == NS - 1)
    def _dump_tail():
        pltpu.sync_copy(agg_sh.at[pl.ds(TAIL0, TAIL)],
                        out_hbm.at[pl.ds(pl.multiple_of(c * N + TAIL0, 8), TAIL)])


@jax.jit
def _sc_aggregate(h, src, dst):
    mesh = plsc.VectorSubcoreMesh(core_axis_name="c", subcore_axis_name="s")
    return pl.kernel(
        _sc_agg_body,
        out_type=jax.ShapeDtypeStruct((NC * N, D), jnp.float32),
        mesh=mesh,
        scratch_types=[
            pltpu.VMEM((CHUNK,), jnp.int32),
            pltpu.VMEM((CHUNK,), jnp.int32),
            pltpu.VMEM((CHUNK, D), jnp.float32),
            pltpu.VMEM_SHARED((N, D), jnp.float32),
            pltpu.SemaphoreType.DMA,
        ],
    )(h, src, dst)


BR = 1000


def _tc_layer_body(p0, p1, h, wr, b, wroot, o):
    agg = p0[...] + p1[...]
    t = lax.dot_general(agg, wr[...], (((1,), (1,)), ((), ())),
                        preferred_element_type=jnp.float32)
    t += lax.dot_general(h[...], wroot[...], (((1,), (1,)), ((), ())),
                         preferred_element_type=jnp.float32)
    o[...] = jnp.maximum(t + b[...], 0.0)


@jax.jit
def _tc_layer(p0, p1, h, wr, b, wroot):
    grid = (N // BR,)
    row_spec = pl.BlockSpec((BR, D), lambda i: (i, 0))
    w_spec = pl.BlockSpec((D, D), lambda i: (0, 0))
    b_spec = pl.BlockSpec((1, D), lambda i: (0, 0))
    return pl.pallas_call(
        _tc_layer_body,
        grid=grid,
        in_specs=[row_spec, row_spec, row_spec, w_spec, b_spec, w_spec],
        out_specs=row_spec,
        out_shape=jax.ShapeDtypeStruct((N, D), jnp.float32),
    )(p0, p1, h, wr, b.reshape(1, D), wroot)


def _tc_final_body(p0, p1, h, wr, b, wroot, wlin, blin, o_h, o_out):
    agg = p0[...] + p1[...]
    t = lax.dot_general(agg, wr[...], (((1,), (1,)), ((), ())),
                        preferred_element_type=jnp.float32)
    t += lax.dot_general(h[...], wroot[...], (((1,), (1,)), ((), ())),
                         preferred_element_type=jnp.float32)
    h3 = jnp.maximum(t + b[...], 0.0)
    o_h[...] = h3
    o_out[...] = lax.dot_general(h3, wlin[...], (((1,), (1,)), ((), ())),
                                 preferred_element_type=jnp.float32) + blin[...]


@jax.jit
def _tc_final(p0, p1, h, wr, b, wroot, wlin, blin):
    grid = (N // BR,)
    row_spec = pl.BlockSpec((BR, D), lambda i: (i, 0))
    w_spec = pl.BlockSpec((D, D), lambda i: (0, 0))
    b_spec = pl.BlockSpec((1, D), lambda i: (0, 0))
    wlin_spec = pl.BlockSpec((D_OUT, D), lambda i: (0, 0))
    blin_spec = pl.BlockSpec((1, D_OUT), lambda i: (0, 0))
    out_spec = pl.BlockSpec((BR, D_OUT), lambda i: (i, 0))
    return pl.pallas_call(
        _tc_final_body,
        grid=grid,
        in_specs=[row_spec, row_spec, row_spec, w_spec, b_spec, w_spec,
                  wlin_spec, blin_spec],
        out_specs=[row_spec, out_spec],
        out_shape=[jax.ShapeDtypeStruct((N, D), jnp.float32),
                   jax.ShapeDtypeStruct((N, D_OUT), jnp.float32)],
    )(p0, p1, h, wr, b.reshape(1, D), wroot, wlin, blin.reshape(1, D_OUT))


def kernel(x, edge_index, W_rel1, b_rel1, W_root1, W_rel2, b_rel2, W_root2,
           W_rel3, b_rel3, W_root3, W_lin, b_lin):
    src = edge_index[0]
    dst = edge_index[1]

    parts = _sc_aggregate(x, src, dst)
    h1 = _tc_layer(parts[:N], parts[N:], x, W_rel1, b_rel1, W_root1)
    parts = _sc_aggregate(h1, src, dst)
    h2 = _tc_layer(parts[:N], parts[N:], h1, W_rel2, b_rel2, W_root2)
    parts = _sc_aggregate(h2, src, dst)
    h3, out = _tc_final(parts[:N], parts[N:], h2, W_rel3, b_rel3, W_root3,
                        W_lin, b_lin)
    return (out, h3)
